# baseline (speedup 1.0000x reference)
import jax
import jax.numpy as jnp
from jax.experimental import pallas as pl
from jax.experimental.pallas import tpu as pltpu


def _conv3x3_nstack(buf, wn, b, sc, sh, h, w):
    cout = wn.shape[1] // 3
    cin = wn.shape[0] // 3
    if buf.shape[1] != cin:
        buf = buf[:, :cin]
    bc = buf if buf.dtype == jnp.bfloat16 else buf.astype(jnp.bfloat16)
    hw = h * w
    catk = jnp.concatenate([bc[0:hw], bc[w:w + hw], bc[2 * w:2 * w + hw]],
                           axis=1)
    p = jnp.dot(catk, wn, preferred_element_type=jnp.float32)
    p0, p1, p2 = p[:, :cout], p[:, cout:2 * cout], p[:, 2 * cout:]
    j = jax.lax.broadcasted_iota(jnp.int32, (hw, cout), 0) % w
    zrow = jnp.zeros((1, cout), jnp.float32)
    p0s = jnp.concatenate([zrow, p0[:-1]], axis=0)
    p2s = jnp.concatenate([p2[1:], zrow], axis=0)
    acc = (p1 + jnp.where(j == 0, 0.0, p0s)
           + jnp.where(j == w - 1, 0.0, p2s))
    y = jnp.maximum(acc + b[...], 0.0)
    return y * sc[...] + sh[...]


def _pad_rows(y, w):
    z = jnp.zeros((w, y.shape[1]), y.dtype)
    return jnp.concatenate([z, y, z], axis=0)


def _maxpool2x2(y, h, w, c, sref):
    sref[...] = y.reshape(h, w, c)
    ho, wo = h // 2, w // 2
    p00 = sref[pl.dslice(0, ho, 2), pl.dslice(0, wo, 2), :]
    p01 = sref[pl.dslice(0, ho, 2), pl.dslice(1, wo, 2), :]
    p10 = sref[pl.dslice(1, ho, 2), pl.dslice(0, wo, 2), :]
    p11 = sref[pl.dslice(1, ho, 2), pl.dslice(1, wo, 2), :]
    p = jnp.maximum(jnp.maximum(p00, p01), jnp.maximum(p10, p11))
    return p.reshape(ho * wo, c)


def _encoder_kernel(x_ref,
                    wk0, wk1, wk2, wk3, wk4, wk5, wk6,
                    b0, s0, t0, b1, s1, t1, b2, s2, t2, b3, s3, t3,
                    b4, s4, t4, b5, s5, t5, b6, s6, t6,
                    o_ref, p1_ref, p2_ref):
    for k in range(x_ref.shape[0]):
        buf = _pad_rows(x_ref[k], 64)
        y = _conv3x3_nstack(buf, wk0[...], b0, s0, t0, 64, 64)
        y = _conv3x3_nstack(_pad_rows(y, 64), wk1[...], b1, s1, t1, 64, 64)
        y = _conv3x3_nstack(_pad_rows(y, 64), wk2[...], b2, s2, t2, 64, 64)
        y = _conv3x3_nstack(_pad_rows(y, 64), wk3[...], b3, s3, t3, 64, 64)
        y = _maxpool2x2(y, 64, 64, 128, p1_ref)
        y = _conv3x3_nstack(_pad_rows(y, 32), wk4[...], b4, s4, t4, 32, 32)
        y = _conv3x3_nstack(_pad_rows(y, 32), wk5[...], b5, s5, t5, 32, 32)
        y = _maxpool2x2(y, 32, 32, 128, p2_ref)
        y = _conv3x3_nstack(_pad_rows(y, 16), wk6[...], b6, s6, t6, 16, 16)
        o_ref[k] = y.T


def kernel(x_nchw, w0, b0, scale0, shift0, w1, b1, scale1, shift1,
           w2, b2, scale2, shift2, w3, b3, scale3, shift3,
           w4, b4, scale4, shift4, w5, b5, scale5, shift5,
           w6, b6, scale6, shift6):
    n = x_nchw.shape[0]
    x = x_nchw.reshape(n, 64 * 64, 1)

    ws = [w0[:, :1, :], w1, w2, w3, w4, w5, w6]

    def _nstack(w, cinp, coutp):
        c_in, c_out = w.shape[1], w.shape[2]
        w4 = w.reshape(3, 3, c_in, c_out)
        w4 = jnp.pad(w4, ((0, 0), (0, 0), (0, cinp - c_in),
                          (0, coutp - c_out)))
        return (w4.transpose(0, 2, 1, 3)
                .reshape(3 * cinp, 3 * coutp).astype(jnp.bfloat16))

    cins = [1, 32, 32, 64, 64, 128, 128]
    wks = [_nstack(ws[i], cins[i], 128) for i in range(6)]
    wks.append(_nstack(ws[6], 128, 256))

    def _padv(v):
        return jnp.pad(v, ((0, 0), (0, 128 - v.shape[1])))
    bss = []
    for trip in ((b0, scale0, shift0), (b1, scale1, shift1),
                 (b2, scale2, shift2), (b3, scale3, shift3)):
        bss += [_padv(v) for v in trip]
    bss += [b4, scale4, shift4, b5, scale5, shift5, b6, scale6, shift6]

    wspecs = [pl.BlockSpec(wk.shape, lambda i: (0, 0)) for wk in wks]
    vspecs = [pl.BlockSpec(v.shape, lambda i: (0, 0)) for v in bss]

    out = pl.pallas_call(
        _encoder_kernel,
        out_shape=jax.ShapeDtypeStruct((n, 256, 256), jnp.float32),
        grid_spec=pltpu.PrefetchScalarGridSpec(
            num_scalar_prefetch=0,
            grid=(n // 8,),
            in_specs=[pl.BlockSpec((8, 64 * 64, 1), lambda i: (i, 0, 0))]
                     + wspecs + vspecs,
            out_specs=pl.BlockSpec((8, 256, 256), lambda i: (i, 0, 0)),
            scratch_shapes=[
                pltpu.VMEM((64, 64, 128), jnp.float32),
                pltpu.VMEM((32, 32, 128), jnp.float32),
            ],
        ),
        compiler_params=pltpu.CompilerParams(
            dimension_semantics=("parallel",)),
    )(x, *wks, *bss)
    return out.reshape(n, 256 * 16 * 16)


# speedup vs baseline: 1.1761x; 1.1761x over previous
import jax
import jax.numpy as jnp
from jax.experimental import pallas as pl
from jax.experimental.pallas import tpu as pltpu


def _conv3x3_nstack(buf, wn, b, sc, sh, h, w):
    cout = wn.shape[1] // 3
    cin = wn.shape[0] // 3
    if buf.shape[1] != cin:
        buf = buf[:, :cin]
    bc = buf if buf.dtype == jnp.bfloat16 else buf.astype(jnp.bfloat16)
    hw = h * w
    catk = jnp.concatenate([bc[0:hw], bc[w:w + hw], bc[2 * w:2 * w + hw]],
                           axis=1)
    p = jnp.dot(catk, wn, preferred_element_type=jnp.float32)
    p0, p1, p2 = p[:, :cout], p[:, cout:2 * cout], p[:, 2 * cout:]
    j = jax.lax.broadcasted_iota(jnp.int32, (hw, cout), 0) % w
    zrow = jnp.zeros((1, cout), jnp.float32)
    p0s = jnp.concatenate([zrow, p0[:-1]], axis=0)
    p2s = jnp.concatenate([p2[1:], zrow], axis=0)
    acc = (p1 + jnp.where(j == 0, 0.0, p0s)
           + jnp.where(j == w - 1, 0.0, p2s))
    y = jnp.maximum(acc + b[...], 0.0)
    return y * sc[...] + sh[...]


def _pad_rows(y, w):
    z = jnp.zeros((w, y.shape[1]), y.dtype)
    return jnp.concatenate([z, y, z], axis=0)


def _maxpool2x2(y, h, w, c, sref):
    sref[...] = y.reshape(h, w, c)
    ho, wo = h // 2, w // 2
    p00 = sref[pl.dslice(0, ho, 2), pl.dslice(0, wo, 2), :]
    p01 = sref[pl.dslice(0, ho, 2), pl.dslice(1, wo, 2), :]
    p10 = sref[pl.dslice(1, ho, 2), pl.dslice(0, wo, 2), :]
    p11 = sref[pl.dslice(1, ho, 2), pl.dslice(1, wo, 2), :]
    p = jnp.maximum(jnp.maximum(p00, p01), jnp.maximum(p10, p11))
    return p.reshape(ho * wo, c)


def _encoder_kernel(x_ref,
                    wk0, wk1, wk2, wk3, wk4, wk5, wk6,
                    b0, s0, t0, b1, s1, t1, b2, s2, t2, b3, s3, t3,
                    b4, s4, t4, b5, s5, t5, b6, s6, t6,
                    o_ref, p1_ref, p2_ref):
    for k in range(x_ref.shape[0]):
        buf = _pad_rows(x_ref[k], 64)
        y = _conv3x3_nstack(buf, wk0[...], b0, s0, t0, 64, 64)
        y = _conv3x3_nstack(_pad_rows(y, 64), wk1[...], b1, s1, t1, 64, 64)
        y = _conv3x3_nstack(_pad_rows(y, 64), wk2[...], b2, s2, t2, 64, 64)
        y = _conv3x3_nstack(_pad_rows(y, 64), wk3[...], b3, s3, t3, 64, 64)
        y = _maxpool2x2(y, 64, 64, 128, p1_ref)
        y = _conv3x3_nstack(_pad_rows(y, 32), wk4[...], b4, s4, t4, 32, 32)
        y = _conv3x3_nstack(_pad_rows(y, 32), wk5[...], b5, s5, t5, 32, 32)
        y = _maxpool2x2(y, 32, 32, 128, p2_ref)
        y = _conv3x3_nstack(_pad_rows(y, 16), wk6[...], b6, s6, t6, 16, 16)
        o_ref[k] = y.T


def kernel(x_nchw, w0, b0, scale0, shift0, w1, b1, scale1, shift1,
           w2, b2, scale2, shift2, w3, b3, scale3, shift3,
           w4, b4, scale4, shift4, w5, b5, scale5, shift5,
           w6, b6, scale6, shift6):
    n = x_nchw.shape[0]
    x = x_nchw.reshape(n, 64 * 64, 1)

    ws = [w0[:, :1, :], w1, w2, w3, w4, w5, w6]

    def _nstack(w, cinp, coutp):
        c_in, c_out = w.shape[1], w.shape[2]
        w4 = w.reshape(3, 3, c_in, c_out)
        w4 = jnp.pad(w4, ((0, 0), (0, 0), (0, cinp - c_in),
                          (0, coutp - c_out)))
        return (w4.transpose(0, 2, 1, 3)
                .reshape(3 * cinp, 3 * coutp).astype(jnp.bfloat16))

    cins = [1, 32, 32, 64, 64, 128, 128]
    wks = [_nstack(ws[i], cins[i], 128) for i in range(6)]
    wks.append(_nstack(ws[6], 128, 256))

    def _padv(v):
        return jnp.pad(v, ((0, 0), (0, 128 - v.shape[1])))
    bss = []
    for trip in ((b0, scale0, shift0), (b1, scale1, shift1),
                 (b2, scale2, shift2), (b3, scale3, shift3)):
        bss += [_padv(v) for v in trip]
    bss += [b4, scale4, shift4, b5, scale5, shift5, b6, scale6, shift6]

    wspecs = [pl.BlockSpec(wk.shape, lambda i: (0, 0)) for wk in wks]
    vspecs = [pl.BlockSpec(v.shape, lambda i: (0, 0)) for v in bss]

    out = pl.pallas_call(
        _encoder_kernel,
        out_shape=jax.ShapeDtypeStruct((n, 256, 256), jnp.float32),
        grid_spec=pltpu.PrefetchScalarGridSpec(
            num_scalar_prefetch=0,
            grid=(n // 4,),
            in_specs=[pl.BlockSpec((4, 64 * 64, 1), lambda i: (i, 0, 0))]
                     + wspecs + vspecs,
            out_specs=pl.BlockSpec((4, 256, 256), lambda i: (i, 0, 0)),
            scratch_shapes=[
                pltpu.VMEM((64, 64, 128), jnp.float32),
                pltpu.VMEM((32, 32, 128), jnp.float32),
            ],
        ),
        compiler_params=pltpu.CompilerParams(
            dimension_semantics=("parallel",)),
    )(x, *wks, *bss)
    return out.reshape(n, 256 * 16 * 16)
